# baseline (speedup 1.0000x reference)
import functools

import jax
import jax.numpy as jnp
from jax import lax
from jax.experimental import pallas as pl
from jax.experimental.pallas import tpu as pltpu
from jax.experimental.pallas import tpu_sc as plsc

F32 = jnp.float32
I32 = jnp.int32

NQ = 8192
BQ = 512
NBLK = NQ // BQ
SCALE_DIMS = ((16, 128, 128), (8, 64, 64), (4, 32, 32))
CHANNELS = (64, 128, 128)

NC, NS = 2, 16
NW = NC * NS
QPW = NQ // NW
CQ = 4
ROWS = CQ * 64
NSTEP = QPW // CQ


def _corner_terms(g, dim):
    ix = ((g + 1.0) * dim - 1.0) * 0.5
    i0 = jnp.floor(ix)
    w1 = ix - i0
    w0 = 1.0 - w1
    i1 = i0 + 1.0
    inb0 = ((i0 >= 0.0) & (i0 <= dim - 1.0)).astype(F32)
    inb1 = ((i1 >= 0.0) & (i1 <= dim - 1.0)).astype(F32)
    c0 = jnp.clip(i0, 0.0, dim - 1.0).astype(I32)
    c1 = jnp.clip(i1, 0.0, dim - 1.0).astype(I32)
    return c0, c1, w0, w1, inb0, inb1


def _prep_body(q_ref, rx_ref, ry_ref, rz_ref,
               ow0, ob0, ww0, wb0, ow1, ob1, ww1, wb1, ow2, ob2, ww2, wb2,
               idx0_o, w0_o, idx1_o, w1_o, idx2_o, w2_o):
    b = pl.program_id(0) // (NBLK // 2)
    q = q_ref[...]
    rx = rx_ref[...]
    ry = ry_ref[...]
    rz = rz_ref[...]
    scales = ((ow0, ob0, ww0, wb0, idx0_o, w0_o),
              (ow1, ob1, ww1, wb1, idx1_o, w1_o),
              (ow2, ob2, ww2, wb2, idx2_o, w2_o))
    for s in range(3):
        ow, ob, ww, wb, idx_o, w_o = scales[s]
        D, H, W = SCALE_DIMS[s]
        off = jnp.tanh(jnp.dot(q, ow[...], preferred_element_type=F32)
                       + ob[...]) * 0.25
        logits = jnp.dot(q, ww[...], preferred_element_type=F32) + wb[...]
        logits = logits - jnp.max(logits, axis=-1, keepdims=True)
        e = jnp.exp(logits)
        attn = e / jnp.sum(e, axis=-1, keepdims=True)
        gx = jnp.clip(rx + off[:, 0:8], 0.0, 1.0) * 2.0 - 1.0
        gy = jnp.clip(ry + off[:, 8:16], 0.0, 1.0) * 2.0 - 1.0
        gz = jnp.clip(rz + off[:, 16:24], 0.0, 1.0) * 2.0 - 1.0
        cx0, cx1, wx0, wx1, ibx0, ibx1 = _corner_terms(gx, float(W))
        cy0, cy1, wy0, wy1, iby0, iby1 = _corner_terms(gy, float(H))
        cz0, cz1, wz0, wz1, ibz0, ibz1 = _corner_terms(gz, float(D))
        idx_cols = []
        w_cols = []
        for cz, czi, wz, ibz in ((0, cz0, wz0, ibz0), (1, cz1, wz1, ibz1)):
            for cy, cyi, wy, iby in ((0, cy0, wy0, iby0), (1, cy1, wy1, iby1)):
                for cx, cxi, wx, ibx in ((0, cx0, wx0, ibx0),
                                         (1, cx1, wx1, ibx1)):
                    idx_cols.append((czi * H + cyi) * W + cxi)
                    w_cols.append(wx * wy * wz * ibx * iby * ibz * attn)
        idx_all = jnp.concatenate(idx_cols, axis=-1) + b * (D * H * W)
        idx_o[...] = idx_all
        w_o[...] = jnp.concatenate(w_cols, axis=-1)


def _full(shape):
    return pl.BlockSpec(shape, lambda i: (0,) * len(shape))


_prep_call = pl.pallas_call(
    _prep_body,
    grid=(NBLK,),
    in_specs=[
        pl.BlockSpec((BQ, 256), lambda i: (i, 0)),
        pl.BlockSpec((BQ, 1), lambda i: (i, 0)),
        pl.BlockSpec((BQ, 1), lambda i: (i, 0)),
        pl.BlockSpec((BQ, 1), lambda i: (i, 0)),
    ] + [_full(s) for s in ((256, 24), (1, 24), (256, 8), (1, 8))] * 3,
    out_specs=[pl.BlockSpec((BQ, 64), lambda i: (i, 0))] * 6,
    out_shape=[jax.ShapeDtypeStruct((NQ, 64), I32),
               jax.ShapeDtypeStruct((NQ, 64), F32)] * 3,
)


def _make_sc_gather(n_rows, C):
    G = C // 32
    mesh = plsc.VectorSubcoreMesh(core_axis_name="c", subcore_axis_name="s",
                                  num_cores=NC)

    @functools.partial(
        pl.kernel, mesh=mesh,
        out_type=jax.ShapeDtypeStruct((NQ, C), F32),
        compiler_params=pltpu.CompilerParams(use_tc_tiling_on_sc=False,
                                             needs_layout_passes=False),
        scratch_types=[
            pltpu.VMEM((2, 2, 128), I32),
            pltpu.VMEM((2, ROWS), F32),
            pltpu.VMEM((2, ROWS, C), jnp.bfloat16),
            pltpu.VMEM((2, CQ, C), F32),
            pltpu.SemaphoreType.DMA,
            pltpu.SemaphoreType.DMA,
        ],
    )
    def sc_gather(table, idx_hbm, w_hbm, out_hbm, idxv, wv, rowsv, outv,
                  gsem_a, gsem_b):
        wid = lax.axis_index("s") * NC + lax.axis_index("c")
        q0 = wid * QPW
        gsems = (gsem_a, gsem_b)

        def load_chunk(i, buf):
            rrow = (q0 + i * CQ) * 64 // 128
            pltpu.sync_copy(idx_hbm.at[pl.ds(rrow, 2)], idxv.at[buf])
            pltpu.sync_copy(w_hbm.at[pl.ds((q0 + i * CQ) * 64, ROWS)],
                            wv.at[buf])

        def fire(buf):
            pltpu.async_copy(table.at[idxv.at[buf, 0]],
                             rowsv.at[buf, pl.ds(0, 128)], gsems[buf])
            pltpu.async_copy(table.at[idxv.at[buf, 1]],
                             rowsv.at[buf, pl.ds(128, 128)], gsems[buf])

        def drain(buf):
            pltpu.make_async_copy(table.at[pl.ds(0, ROWS)],
                                  rowsv.at[buf], gsems[buf]).wait()

        def compute(i, buf):
            def qstep(qq, carry):
                def jstep(j16, acc):
                    rb = qq * 64 + j16 * 16
                    wvec = wv[buf, pl.ds(rb, 16)]
                    acc = list(acc)
                    for jj in range(16):
                        wj = wvec[jj]
                        for g in range(G):
                            ab = rowsv[buf, rb + jj, pl.ds(g * 32, 32)]
                            w32 = plsc.bitcast(ab, I32)
                            ev = plsc.bitcast(w32 << 16, F32)
                            od = plsc.bitcast(w32 & jnp.int32(-65536), F32)
                            acc[2 * g] = acc[2 * g] + wj * ev
                            acc[2 * g + 1] = acc[2 * g + 1] + wj * od
                    return tuple(acc)
                acc = plsc.parallel_loop(
                    0, 4, unroll=2,
                    carry=tuple(jnp.zeros((16,), F32)
                                for _ in range(2 * G)))(jstep)
                for c in range(2 * G):
                    outv[buf, qq, pl.ds(c * 16, 16)] = acc[c]
                return carry
            lax.fori_loop(0, CQ, qstep, 0)
            pltpu.sync_copy(outv.at[buf],
                            out_hbm.at[pl.ds(q0 + i * CQ, CQ)])

        load_chunk(0, 0)
        fire(0)
        load_chunk(1, 1)
        fire(1)

        def step(t, carry):
            i = 2 * t
            drain(0)
            compute(i, 0)
            load_chunk(i + 2, 0)
            fire(0)
            drain(1)
            compute(i + 1, 1)
            load_chunk(i + 3, 1)
            fire(1)
            return carry

        lax.fori_loop(0, NSTEP // 2 - 1, step, 0)
        drain(0)
        compute(NSTEP - 2, 0)
        drain(1)
        compute(NSTEP - 1, 1)

    return sc_gather


@functools.lru_cache(maxsize=None)
def _sc_gather_call(s):
    D, H, W = SCALE_DIMS[s]
    return _make_sc_gather(2 * D * H * W, CHANNELS[s])


def _proj_body(g0_ref, g1_ref, g2_ref,
               vw0, vb0, vw1, vb1, vw2, vb2, lng, lnb, outw, outb, o_ref):
    a0 = jnp.dot(g0_ref[...], vw0[...], preferred_element_type=F32) + vb0[...]
    a1 = jnp.dot(g1_ref[...], vw1[...], preferred_element_type=F32) + vb1[...]
    a2 = jnp.dot(g2_ref[...], vw2[...], preferred_element_type=F32) + vb2[...]
    cat = jnp.concatenate([a0, a1, a2], axis=-1)
    mu = jnp.mean(cat, axis=-1, keepdims=True)
    var = jnp.mean((cat - mu) ** 2, axis=-1, keepdims=True)
    ln = (cat - mu) * jax.lax.rsqrt(var + 1e-5) * lng[...] + lnb[...]
    o_ref[...] = jnp.dot(ln, outw[...], preferred_element_type=F32) + outb[...]


_proj_call = pl.pallas_call(
    _proj_body,
    grid=(NBLK,),
    in_specs=[
        pl.BlockSpec((BQ, 64), lambda i: (i, 0)),
        pl.BlockSpec((BQ, 128), lambda i: (i, 0)),
        pl.BlockSpec((BQ, 128), lambda i: (i, 0)),
        _full((64, 256)), _full((1, 256)),
        _full((128, 256)), _full((1, 256)),
        _full((128, 256)), _full((1, 256)),
        _full((1, 768)), _full((1, 768)),
        _full((768, 256)), _full((1, 256)),
    ],
    out_specs=pl.BlockSpec((BQ, 256), lambda i: (i, 0)),
    out_shape=jax.ShapeDtypeStruct((NQ, 256), F32),
)


def _xpose_body(v_ref, o_ref):
    o_ref[...] = jnp.swapaxes(v_ref[0], 0, 1)[None].astype(jnp.bfloat16)


@functools.lru_cache(maxsize=None)
def _xpose_call(s):
    D, H, W = SCALE_DIMS[s]
    C = CHANNELS[s]
    DHW = D * H * W
    CH_ROWS = 2048
    K = DHW // CH_ROWS
    return pl.pallas_call(
        _xpose_body,
        grid=(2, K),
        in_specs=[pl.BlockSpec((1, C, CH_ROWS), lambda b, k: (b, 0, k))],
        out_specs=pl.BlockSpec((1, CH_ROWS, C), lambda b, k: (b, k, 0)),
        out_shape=jax.ShapeDtypeStruct((2, DHW, C), jnp.bfloat16),
    )


def _unpack_perm(C):
    perm = []
    for g in range(C // 32):
        perm += [32 * g + 2 * k for k in range(16)]
        perm += [32 * g + 2 * k + 1 for k in range(16)]
    return perm


def _reorder_off(w):
    return w.reshape(-1, 8, 3).swapaxes(-1, -2).reshape(w.shape[:-1] + (24,))


def kernel(queries, reference_points, occ_vol_0, occ_vol_1, occ_vol_2,
           off_w0, off_b0, wt_w0, wt_b0, val_w0, val_b0,
           off_w1, off_b1, wt_w1, wt_b1, val_w1, val_b1,
           off_w2, off_b2, wt_w2, wt_b2, val_w2, val_b2,
           ln_g, ln_b, out_w, out_b):
    q2d = queries.reshape(NQ, 256)
    ref2d = reference_points.reshape(NQ, 3)
    rx, ry, rz = ref2d[:, 0:1], ref2d[:, 1:2], ref2d[:, 2:3]
    prep_args = [q2d, rx, ry, rz]
    for ow, ob, ww, wb in ((off_w0, off_b0, wt_w0, wt_b0),
                           (off_w1, off_b1, wt_w1, wt_b1),
                           (off_w2, off_b2, wt_w2, wt_b2)):
        prep_args += [_reorder_off(ow), _reorder_off(ob.reshape(1, 24)),
                      ww, wb.reshape(1, 8)]
    idx0, w0, idx1, w1, idx2, w2 = _prep_call(*prep_args)

    gathered = []
    for s, (vol, idx, w) in enumerate(((occ_vol_0, idx0, w0),
                                       (occ_vol_1, idx1, w1),
                                       (occ_vol_2, idx2, w2))):
        D, H, W = SCALE_DIMS[s]
        C = CHANNELS[s]
        table = _xpose_call(s)(vol.reshape(2, C, D * H * W))
        table = table.reshape(2 * D * H * W, C)
        gathered.append(_sc_gather_call(s)(table, idx.reshape(-1, 128),
                                           w.reshape(-1)))

    out = _proj_call(gathered[0], gathered[1], gathered[2],
                     val_w0[_unpack_perm(64), :], val_b0.reshape(1, 256),
                     val_w1[_unpack_perm(128), :], val_b1.reshape(1, 256),
                     val_w2[_unpack_perm(128), :], val_b2.reshape(1, 256),
                     ln_g.reshape(1, 768), ln_b.reshape(1, 768),
                     out_w, out_b.reshape(1, 256))
    return out.reshape(2, 4096, 256)


# speedup vs baseline: 1.2289x; 1.2289x over previous
import functools

import jax
import jax.numpy as jnp
from jax import lax
from jax.experimental import pallas as pl
from jax.experimental.pallas import tpu as pltpu
from jax.experimental.pallas import tpu_sc as plsc

F32 = jnp.float32
I32 = jnp.int32

NQ = 8192
BQ = 512
NBLK = NQ // BQ
SCALE_DIMS = ((16, 128, 128), (8, 64, 64), (4, 32, 32))
CHANNELS = (64, 128, 128)

NC, NS = 2, 16
NW = NC * NS
QPW = NQ // NW
CQ = 4
ROWS = CQ * 64
NSTEP = QPW // CQ


def _corner_terms(g, dim):
    ix = ((g + 1.0) * dim - 1.0) * 0.5
    i0 = jnp.floor(ix)
    w1 = ix - i0
    w0 = 1.0 - w1
    i1 = i0 + 1.0
    inb0 = ((i0 >= 0.0) & (i0 <= dim - 1.0)).astype(F32)
    inb1 = ((i1 >= 0.0) & (i1 <= dim - 1.0)).astype(F32)
    c0 = jnp.clip(i0, 0.0, dim - 1.0).astype(I32)
    c1 = jnp.clip(i1, 0.0, dim - 1.0).astype(I32)
    return c0, c1, w0, w1, inb0, inb1


def _prep_body(q_ref, rx_ref, ry_ref, rz_ref,
               ow0, ob0, ww0, wb0, ow1, ob1, ww1, wb1, ow2, ob2, ww2, wb2,
               idx0_o, w0_o, idx1_o, w1_o, idx2_o, w2_o):
    b = pl.program_id(0) // (NBLK // 2)
    q = q_ref[...]
    rx = rx_ref[...]
    ry = ry_ref[...]
    rz = rz_ref[...]
    scales = ((ow0, ob0, ww0, wb0, idx0_o, w0_o),
              (ow1, ob1, ww1, wb1, idx1_o, w1_o),
              (ow2, ob2, ww2, wb2, idx2_o, w2_o))
    for s in range(3):
        ow, ob, ww, wb, idx_o, w_o = scales[s]
        D, H, W = SCALE_DIMS[s]
        off = jnp.tanh(jnp.dot(q, ow[...], preferred_element_type=F32)
                       + ob[...]) * 0.25
        logits = jnp.dot(q, ww[...], preferred_element_type=F32) + wb[...]
        logits = logits - jnp.max(logits, axis=-1, keepdims=True)
        e = jnp.exp(logits)
        attn = e / jnp.sum(e, axis=-1, keepdims=True)
        gx = jnp.clip(rx + off[:, 0:8], 0.0, 1.0) * 2.0 - 1.0
        gy = jnp.clip(ry + off[:, 8:16], 0.0, 1.0) * 2.0 - 1.0
        gz = jnp.clip(rz + off[:, 16:24], 0.0, 1.0) * 2.0 - 1.0
        cx0, cx1, wx0, wx1, ibx0, ibx1 = _corner_terms(gx, float(W))
        cy0, cy1, wy0, wy1, iby0, iby1 = _corner_terms(gy, float(H))
        cz0, cz1, wz0, wz1, ibz0, ibz1 = _corner_terms(gz, float(D))
        idx_cols = []
        w_cols = []
        for cz, czi, wz, ibz in ((0, cz0, wz0, ibz0), (1, cz1, wz1, ibz1)):
            for cy, cyi, wy, iby in ((0, cy0, wy0, iby0), (1, cy1, wy1, iby1)):
                for cx, cxi, wx, ibx in ((0, cx0, wx0, ibx0),
                                         (1, cx1, wx1, ibx1)):
                    idx_cols.append((czi * H + cyi) * W + cxi)
                    w_cols.append(wx * wy * wz * ibx * iby * ibz * attn)
        idx_all = jnp.concatenate(idx_cols, axis=-1) + b * (D * H * W)
        idx_o[...] = idx_all
        w_o[...] = jnp.concatenate(w_cols, axis=-1)


def _full(shape):
    return pl.BlockSpec(shape, lambda i: (0,) * len(shape))


_prep_call = pl.pallas_call(
    _prep_body,
    grid=(NBLK,),
    in_specs=[
        pl.BlockSpec((BQ, 256), lambda i: (i, 0)),
        pl.BlockSpec((BQ, 1), lambda i: (i, 0)),
        pl.BlockSpec((BQ, 1), lambda i: (i, 0)),
        pl.BlockSpec((BQ, 1), lambda i: (i, 0)),
    ] + [_full(s) for s in ((256, 24), (1, 24), (256, 8), (1, 8))] * 3,
    out_specs=[pl.BlockSpec((BQ, 64), lambda i: (i, 0))] * 6,
    out_shape=[jax.ShapeDtypeStruct((NQ, 64), I32),
               jax.ShapeDtypeStruct((NQ, 64), F32)] * 3,
)


def _make_sc_gather(n_rows, C):
    G = C // 32
    mesh = plsc.VectorSubcoreMesh(core_axis_name="c", subcore_axis_name="s",
                                  num_cores=NC)

    @functools.partial(
        pl.kernel, mesh=mesh,
        out_type=jax.ShapeDtypeStruct((NQ, C), F32),
        compiler_params=pltpu.CompilerParams(use_tc_tiling_on_sc=False,
                                             needs_layout_passes=False),
        scratch_types=[
            pltpu.VMEM((2, 2, 128), I32),
            pltpu.VMEM((2, ROWS), F32),
            pltpu.VMEM((2, ROWS, C), jnp.bfloat16),
            pltpu.VMEM((2, CQ, C), F32),
            pltpu.SemaphoreType.DMA,
            pltpu.SemaphoreType.DMA,
        ],
    )
    def sc_gather(table, idx_hbm, w_hbm, out_hbm, idxv, wv, rowsv, outv,
                  gsem_a, gsem_b):
        wid = lax.axis_index("s") * NC + lax.axis_index("c")
        q0 = wid * QPW
        gsems = (gsem_a, gsem_b)

        def load_chunk(i, buf):
            rrow = (q0 + i * CQ) * 64 // 128
            pltpu.sync_copy(idx_hbm.at[pl.ds(rrow, 2)], idxv.at[buf])
            pltpu.sync_copy(w_hbm.at[pl.ds((q0 + i * CQ) * 64, ROWS)],
                            wv.at[buf])

        def fire(buf):
            pltpu.async_copy(table.at[idxv.at[buf, 0]],
                             rowsv.at[buf, pl.ds(0, 128)], gsems[buf])
            pltpu.async_copy(table.at[idxv.at[buf, 1]],
                             rowsv.at[buf, pl.ds(128, 128)], gsems[buf])

        def drain(buf):
            pltpu.make_async_copy(table.at[pl.ds(0, ROWS)],
                                  rowsv.at[buf], gsems[buf]).wait()

        def compute(i, buf):
            def qstep(qq, carry):
                def jstep(j16, acc):
                    rb = qq * 64 + j16 * 16
                    wvec = wv[buf, pl.ds(rb, 16)]
                    acc = list(acc)
                    for jj in range(16):
                        wj = wvec[jj]
                        for g in range(G):
                            ab = rowsv[buf, rb + jj, pl.ds(g * 32, 32)]
                            w32 = plsc.bitcast(ab, I32)
                            ev = plsc.bitcast(w32 << 16, F32)
                            od = plsc.bitcast(w32 & jnp.int32(-65536), F32)
                            acc[2 * g] = acc[2 * g] + wj * ev
                            acc[2 * g + 1] = acc[2 * g + 1] + wj * od
                    return tuple(acc)
                acc = plsc.parallel_loop(
                    0, 4, unroll=2,
                    carry=tuple(jnp.zeros((16,), F32)
                                for _ in range(2 * G)))(jstep)
                for c in range(2 * G):
                    outv[buf, qq, pl.ds(c * 16, 16)] = acc[c]
                return carry
            lax.fori_loop(0, CQ, qstep, 0)
            pltpu.sync_copy(outv.at[buf],
                            out_hbm.at[pl.ds(q0 + i * CQ, CQ)])

        load_chunk(0, 0)
        fire(0)
        load_chunk(1, 1)
        fire(1)

        def step(t, carry):
            i = 2 * t
            drain(0)
            compute(i, 0)
            load_chunk(i + 2, 0)
            fire(0)
            drain(1)
            compute(i + 1, 1)
            load_chunk(i + 3, 1)
            fire(1)
            return carry

        lax.fori_loop(0, NSTEP // 2 - 1, step, 0)
        drain(0)
        compute(NSTEP - 2, 0)
        drain(1)
        compute(NSTEP - 1, 1)

    return sc_gather


@functools.lru_cache(maxsize=None)
def _sc_gather_call(s):
    D, H, W = SCALE_DIMS[s]
    return _make_sc_gather(2 * D * H * W, CHANNELS[s])


def _proj_body(g0_ref, g1_ref, g2_ref,
               vw0, vb0, vw1, vb1, vw2, vb2, lng, lnb, outw, outb, o_ref):
    a0 = jnp.dot(g0_ref[...], vw0[...], preferred_element_type=F32) + vb0[...]
    a1 = jnp.dot(g1_ref[...], vw1[...], preferred_element_type=F32) + vb1[...]
    a2 = jnp.dot(g2_ref[...], vw2[...], preferred_element_type=F32) + vb2[...]
    cat = jnp.concatenate([a0, a1, a2], axis=-1)
    mu = jnp.mean(cat, axis=-1, keepdims=True)
    var = jnp.mean((cat - mu) ** 2, axis=-1, keepdims=True)
    ln = (cat - mu) * jax.lax.rsqrt(var + 1e-5) * lng[...] + lnb[...]
    o_ref[...] = jnp.dot(ln, outw[...], preferred_element_type=F32) + outb[...]


_proj_call = pl.pallas_call(
    _proj_body,
    grid=(NBLK,),
    in_specs=[
        pl.BlockSpec((BQ, 64), lambda i: (i, 0)),
        pl.BlockSpec((BQ, 128), lambda i: (i, 0)),
        pl.BlockSpec((BQ, 128), lambda i: (i, 0)),
        _full((64, 256)), _full((1, 256)),
        _full((128, 256)), _full((1, 256)),
        _full((128, 256)), _full((1, 256)),
        _full((1, 768)), _full((1, 768)),
        _full((768, 256)), _full((1, 256)),
    ],
    out_specs=pl.BlockSpec((BQ, 256), lambda i: (i, 0)),
    out_shape=jax.ShapeDtypeStruct((NQ, 256), F32),
)


def _xpose_body(v_ref, o_ref):
    o_ref[...] = jnp.swapaxes(v_ref[0], 0, 1)[None].astype(jnp.bfloat16)


@functools.lru_cache(maxsize=None)
def _xpose_call(s):
    D, H, W = SCALE_DIMS[s]
    C = CHANNELS[s]
    DHW = D * H * W
    CH_ROWS = 2048
    K = DHW // CH_ROWS
    return pl.pallas_call(
        _xpose_body,
        grid=(2, K),
        in_specs=[pl.BlockSpec((1, C, CH_ROWS), lambda b, k: (b, 0, k))],
        out_specs=pl.BlockSpec((1, CH_ROWS, C), lambda b, k: (b, k, 0)),
        out_shape=jax.ShapeDtypeStruct((2, DHW, C), jnp.bfloat16),
    )


def _unpack_perm(C):
    perm = []
    for g in range(C // 32):
        perm += [32 * g + 2 * k for k in range(16)]
        perm += [32 * g + 2 * k + 1 for k in range(16)]
    return perm


def _reorder_off(w):
    return w.reshape(-1, 8, 3).swapaxes(-1, -2).reshape(w.shape[:-1] + (24,))


def kernel(queries, reference_points, occ_vol_0, occ_vol_1, occ_vol_2,
           off_w0, off_b0, wt_w0, wt_b0, val_w0, val_b0,
           off_w1, off_b1, wt_w1, wt_b1, val_w1, val_b1,
           off_w2, off_b2, wt_w2, wt_b2, val_w2, val_b2,
           ln_g, ln_b, out_w, out_b):
    q2d = queries.reshape(NQ, 256)
    ref2d = reference_points.reshape(NQ, 3)
    rx, ry, rz = ref2d[:, 0:1], ref2d[:, 1:2], ref2d[:, 2:3]
    prep_args = [q2d, rx, ry, rz]
    for ow, ob, ww, wb in ((off_w0, off_b0, wt_w0, wt_b0),
                           (off_w1, off_b1, wt_w1, wt_b1),
                           (off_w2, off_b2, wt_w2, wt_b2)):
        prep_args += [_reorder_off(ow), _reorder_off(ob.reshape(1, 24)),
                      ww, wb.reshape(1, 8)]
    idx0, w0, idx1, w1, idx2, w2 = _prep_call(*prep_args)

    gathered = []
    for s, (vol, idx, w) in enumerate(((occ_vol_0, idx0, w0),
                                       (occ_vol_1, idx1, w1),
                                       (occ_vol_2, idx2, w2))):
        D, H, W = SCALE_DIMS[s]
        C = CHANNELS[s]
        table = vol.transpose(0, 2, 3, 4, 1).reshape(
            2 * D * H * W, C).astype(jnp.bfloat16)
        gathered.append(_sc_gather_call(s)(table, idx.reshape(-1, 128),
                                           w.reshape(-1)))

    out = _proj_call(gathered[0], gathered[1], gathered[2],
                     val_w0[_unpack_perm(64), :], val_b0.reshape(1, 256),
                     val_w1[_unpack_perm(128), :], val_b1.reshape(1, 256),
                     val_w2[_unpack_perm(128), :], val_b2.reshape(1, 256),
                     ln_g.reshape(1, 768), ln_b.reshape(1, 768),
                     out_w, out_b.reshape(1, 256))
    return out.reshape(2, 4096, 256)


# speedup vs baseline: 1.2302x; 1.0010x over previous
import functools

import jax
import jax.numpy as jnp
from jax import lax
from jax.experimental import pallas as pl
from jax.experimental.pallas import tpu as pltpu
from jax.experimental.pallas import tpu_sc as plsc

F32 = jnp.float32
I32 = jnp.int32

NQ = 8192
BQ = 512
NBLK = NQ // BQ
SCALE_DIMS = ((16, 128, 128), (8, 64, 64), (4, 32, 32))
CHANNELS = (64, 128, 128)

NC, NS = 2, 16
NW = NC * NS
QPW = NQ // NW
CQ = 4
ROWS = CQ * 64
NSTEP = QPW // CQ


def _corner_terms(g, dim):
    ix = ((g + 1.0) * dim - 1.0) * 0.5
    i0 = jnp.floor(ix)
    w1 = ix - i0
    w0 = 1.0 - w1
    i1 = i0 + 1.0
    inb0 = ((i0 >= 0.0) & (i0 <= dim - 1.0)).astype(F32)
    inb1 = ((i1 >= 0.0) & (i1 <= dim - 1.0)).astype(F32)
    c0 = jnp.clip(i0, 0.0, dim - 1.0).astype(I32)
    c1 = jnp.clip(i1, 0.0, dim - 1.0).astype(I32)
    return c0, c1, w0, w1, inb0, inb1


def _prep_body(q_ref, rx_ref, ry_ref, rz_ref,
               ow0, ob0, ww0, wb0, ow1, ob1, ww1, wb1, ow2, ob2, ww2, wb2,
               idx0_o, w0_o, idx1_o, w1_o, idx2_o, w2_o):
    b = pl.program_id(0) // (NBLK // 2)
    q = q_ref[...]
    rx = rx_ref[...]
    ry = ry_ref[...]
    rz = rz_ref[...]
    scales = ((ow0, ob0, ww0, wb0, idx0_o, w0_o),
              (ow1, ob1, ww1, wb1, idx1_o, w1_o),
              (ow2, ob2, ww2, wb2, idx2_o, w2_o))
    for s in range(3):
        ow, ob, ww, wb, idx_o, w_o = scales[s]
        D, H, W = SCALE_DIMS[s]
        off = jnp.tanh(jnp.dot(q, ow[...], preferred_element_type=F32)
                       + ob[...]) * 0.25
        logits = jnp.dot(q, ww[...], preferred_element_type=F32) + wb[...]
        logits = logits - jnp.max(logits, axis=-1, keepdims=True)
        e = jnp.exp(logits)
        attn = e / jnp.sum(e, axis=-1, keepdims=True)
        gx = jnp.clip(rx + off[:, 0:8], 0.0, 1.0) * 2.0 - 1.0
        gy = jnp.clip(ry + off[:, 8:16], 0.0, 1.0) * 2.0 - 1.0
        gz = jnp.clip(rz + off[:, 16:24], 0.0, 1.0) * 2.0 - 1.0
        cx0, cx1, wx0, wx1, ibx0, ibx1 = _corner_terms(gx, float(W))
        cy0, cy1, wy0, wy1, iby0, iby1 = _corner_terms(gy, float(H))
        cz0, cz1, wz0, wz1, ibz0, ibz1 = _corner_terms(gz, float(D))
        idx_cols = []
        w_cols = []
        for cz, czi, wz, ibz in ((0, cz0, wz0, ibz0), (1, cz1, wz1, ibz1)):
            for cy, cyi, wy, iby in ((0, cy0, wy0, iby0), (1, cy1, wy1, iby1)):
                for cx, cxi, wx, ibx in ((0, cx0, wx0, ibx0),
                                         (1, cx1, wx1, ibx1)):
                    idx_cols.append((czi * H + cyi) * W + cxi)
                    w_cols.append(wx * wy * wz * ibx * iby * ibz * attn)
        idx_all = jnp.concatenate(idx_cols, axis=-1) + b * (D * H * W)
        idx_o[...] = idx_all
        w_o[...] = jnp.concatenate(w_cols, axis=-1)


def _full(shape):
    return pl.BlockSpec(shape, lambda i: (0,) * len(shape))


_prep_call = pl.pallas_call(
    _prep_body,
    grid=(NBLK,),
    in_specs=[
        pl.BlockSpec((BQ, 256), lambda i: (i, 0)),
        pl.BlockSpec((BQ, 1), lambda i: (i, 0)),
        pl.BlockSpec((BQ, 1), lambda i: (i, 0)),
        pl.BlockSpec((BQ, 1), lambda i: (i, 0)),
    ] + [_full(s) for s in ((256, 24), (1, 24), (256, 8), (1, 8))] * 3,
    out_specs=[pl.BlockSpec((BQ, 64), lambda i: (i, 0))] * 6,
    out_shape=[jax.ShapeDtypeStruct((NQ, 64), I32),
               jax.ShapeDtypeStruct((NQ, 64), F32)] * 3,
)


def _make_sc_gather(n_rows, C):
    G = C // 32
    mesh = plsc.VectorSubcoreMesh(core_axis_name="c", subcore_axis_name="s",
                                  num_cores=NC)

    @functools.partial(
        pl.kernel, mesh=mesh,
        out_type=jax.ShapeDtypeStruct((NQ, C), F32),
        compiler_params=pltpu.CompilerParams(use_tc_tiling_on_sc=False,
                                             needs_layout_passes=False),
        scratch_types=[
            pltpu.VMEM((2, 2, 128), I32),
            pltpu.VMEM((2, ROWS), F32),
            pltpu.VMEM((2, ROWS, C), jnp.bfloat16),
            pltpu.VMEM((2, CQ, C), F32),
            pltpu.SemaphoreType.DMA,
            pltpu.SemaphoreType.DMA,
        ],
    )
    def sc_gather(table, idx_hbm, w_hbm, out_hbm, idxv, wv, rowsv, outv,
                  gsem_a, gsem_b):
        wid = lax.axis_index("s") * NC + lax.axis_index("c")
        q0 = wid * QPW
        gsems = (gsem_a, gsem_b)

        def load_chunk(i, buf):
            rrow = (q0 + i * CQ) * 64 // 128
            pltpu.sync_copy(idx_hbm.at[pl.ds(rrow, 2)], idxv.at[buf])
            pltpu.sync_copy(w_hbm.at[pl.ds((q0 + i * CQ) * 64, ROWS)],
                            wv.at[buf])

        def fire(buf):
            pltpu.async_copy(table.at[idxv.at[buf, 0]],
                             rowsv.at[buf, pl.ds(0, 128)], gsems[buf])
            pltpu.async_copy(table.at[idxv.at[buf, 1]],
                             rowsv.at[buf, pl.ds(128, 128)], gsems[buf])

        def drain(buf):
            pltpu.make_async_copy(table.at[pl.ds(0, ROWS)],
                                  rowsv.at[buf], gsems[buf]).wait()

        def compute(i, buf):
            def qstep(qq, carry):
                def jstep(j16, acc):
                    rb = qq * 64 + j16 * 16
                    wvec = wv[buf, pl.ds(rb, 16)]
                    acc = list(acc)
                    for jj in range(16):
                        wj = wvec[jj]
                        for g in range(G):
                            ab = rowsv[buf, rb + jj, pl.ds(g * 32, 32)]
                            w32 = plsc.bitcast(ab, I32)
                            ev = plsc.bitcast(w32 << 16, F32)
                            od = plsc.bitcast(w32 & jnp.int32(-65536), F32)
                            acc[2 * g] = acc[2 * g] + wj * ev
                            acc[2 * g + 1] = acc[2 * g + 1] + wj * od
                    return tuple(acc)
                acc = plsc.parallel_loop(
                    0, 4, unroll=2,
                    carry=tuple(jnp.zeros((16,), F32)
                                for _ in range(2 * G)))(jstep)
                for c in range(2 * G):
                    outv[buf, qq, pl.ds(c * 16, 16)] = acc[c]
                return carry
            lax.fori_loop(0, CQ, qstep, 0)
            pltpu.sync_copy(outv.at[buf],
                            out_hbm.at[pl.ds(q0 + i * CQ, CQ)])

        load_chunk(0, 0)
        fire(0)
        load_chunk(1, 1)
        fire(1)

        def step(t, carry):
            i = 2 * t
            drain(0)
            compute(i, 0)
            load_chunk(i + 2, 0)
            fire(0)
            drain(1)
            compute(i + 1, 1)
            load_chunk(i + 3, 1)
            fire(1)
            return carry

        lax.fori_loop(0, NSTEP // 2 - 1, step, 0)
        drain(0)
        compute(NSTEP - 2, 0)
        drain(1)
        compute(NSTEP - 1, 1)

    return sc_gather


@functools.lru_cache(maxsize=None)
def _sc_gather_call(s):
    D, H, W = SCALE_DIMS[s]
    return _make_sc_gather(2 * D * H * W, CHANNELS[s])


def _proj_body(g0_ref, g1_ref, g2_ref,
               vw0, vb0, vw1, vb1, vw2, vb2, lng, lnb, outw, outb, o_ref):
    a0 = jnp.dot(g0_ref[...], vw0[...], preferred_element_type=F32) + vb0[...]
    a1 = jnp.dot(g1_ref[...], vw1[...], preferred_element_type=F32) + vb1[...]
    a2 = jnp.dot(g2_ref[...], vw2[...], preferred_element_type=F32) + vb2[...]
    cat = jnp.concatenate([a0, a1, a2], axis=-1)
    mu = jnp.mean(cat, axis=-1, keepdims=True)
    var = jnp.mean((cat - mu) ** 2, axis=-1, keepdims=True)
    ln = (cat - mu) * jax.lax.rsqrt(var + 1e-5) * lng[...] + lnb[...]
    o_ref[...] = jnp.dot(ln, outw[...], preferred_element_type=F32) + outb[...]


_proj_call = pl.pallas_call(
    _proj_body,
    grid=(NBLK,),
    in_specs=[
        pl.BlockSpec((BQ, 64), lambda i: (i, 0)),
        pl.BlockSpec((BQ, 128), lambda i: (i, 0)),
        pl.BlockSpec((BQ, 128), lambda i: (i, 0)),
        _full((64, 256)), _full((1, 256)),
        _full((128, 256)), _full((1, 256)),
        _full((128, 256)), _full((1, 256)),
        _full((1, 768)), _full((1, 768)),
        _full((768, 256)), _full((1, 256)),
    ],
    out_specs=pl.BlockSpec((BQ, 256), lambda i: (i, 0)),
    out_shape=jax.ShapeDtypeStruct((NQ, 256), F32),
)


def _xpose_body(v_ref, o_ref):
    o_ref[...] = jnp.swapaxes(v_ref[0], 0, 1)[None].astype(jnp.bfloat16)


@functools.lru_cache(maxsize=None)
def _xpose_call(s):
    D, H, W = SCALE_DIMS[s]
    C = CHANNELS[s]
    DHW = D * H * W
    CH_ROWS = 2048
    K = DHW // CH_ROWS
    return pl.pallas_call(
        _xpose_body,
        grid=(2, K),
        in_specs=[pl.BlockSpec((1, C, CH_ROWS), lambda b, k: (b, 0, k))],
        out_specs=pl.BlockSpec((1, CH_ROWS, C), lambda b, k: (b, k, 0)),
        out_shape=jax.ShapeDtypeStruct((2, DHW, C), jnp.bfloat16),
    )


def _unpack_perm(C):
    perm = []
    for g in range(C // 32):
        perm += [32 * g + 2 * k for k in range(16)]
        perm += [32 * g + 2 * k + 1 for k in range(16)]
    return perm


def _reorder_off(w):
    return w.reshape(-1, 8, 3).swapaxes(-1, -2).reshape(w.shape[:-1] + (24,))


def kernel(queries, reference_points, occ_vol_0, occ_vol_1, occ_vol_2,
           off_w0, off_b0, wt_w0, wt_b0, val_w0, val_b0,
           off_w1, off_b1, wt_w1, wt_b1, val_w1, val_b1,
           off_w2, off_b2, wt_w2, wt_b2, val_w2, val_b2,
           ln_g, ln_b, out_w, out_b):
    q2d = queries.reshape(NQ, 256)
    ref2d = reference_points.reshape(NQ, 3)
    rx, ry, rz = ref2d[:, 0:1], ref2d[:, 1:2], ref2d[:, 2:3]
    prep_args = [q2d, rx, ry, rz]
    for ow, ob, ww, wb in ((off_w0, off_b0, wt_w0, wt_b0),
                           (off_w1, off_b1, wt_w1, wt_b1),
                           (off_w2, off_b2, wt_w2, wt_b2)):
        prep_args += [_reorder_off(ow), _reorder_off(ob.reshape(1, 24)),
                      ww, wb.reshape(1, 8)]
    idx0, w0, idx1, w1, idx2, w2 = _prep_call(*prep_args)

    gathered = []
    for s, (vol, idx, w) in enumerate(((occ_vol_0, idx0, w0),
                                       (occ_vol_1, idx1, w1),
                                       (occ_vol_2, idx2, w2))):
        D, H, W = SCALE_DIMS[s]
        C = CHANNELS[s]
        table = vol.astype(jnp.bfloat16).transpose(0, 2, 3, 4, 1).reshape(
            2 * D * H * W, C)
        gathered.append(_sc_gather_call(s)(table, idx.reshape(-1, 128),
                                           w.reshape(-1)))

    out = _proj_call(gathered[0], gathered[1], gathered[2],
                     val_w0[_unpack_perm(64), :], val_b0.reshape(1, 256),
                     val_w1[_unpack_perm(128), :], val_b1.reshape(1, 256),
                     val_w2[_unpack_perm(128), :], val_b2.reshape(1, 256),
                     ln_g.reshape(1, 768), ln_b.reshape(1, 768),
                     out_w, out_b.reshape(1, 256))
    return out.reshape(2, 4096, 256)


# speedup vs baseline: 1.4083x; 1.1448x over previous
import functools

import jax
import jax.numpy as jnp
from jax import lax
from jax.experimental import pallas as pl
from jax.experimental.pallas import tpu as pltpu
from jax.experimental.pallas import tpu_sc as plsc

F32 = jnp.float32
I32 = jnp.int32

NQ = 8192
BQ = 512
NBLK = NQ // BQ
SCALE_DIMS = ((16, 128, 128), (8, 64, 64), (4, 32, 32))
CHANNELS = (64, 128, 128)

NC, NS = 2, 16
NW = NC * NS
QPW = NQ // NW
CQ = 4
ROWS = CQ * 64
NSTEP = QPW // CQ


def _corner_terms(g, dim):
    ix = ((g + 1.0) * dim - 1.0) * 0.5
    i0 = jnp.floor(ix)
    w1 = ix - i0
    w0 = 1.0 - w1
    i1 = i0 + 1.0
    inb0 = ((i0 >= 0.0) & (i0 <= dim - 1.0)).astype(F32)
    inb1 = ((i1 >= 0.0) & (i1 <= dim - 1.0)).astype(F32)
    c0 = jnp.clip(i0, 0.0, dim - 1.0).astype(I32)
    c1 = jnp.clip(i1, 0.0, dim - 1.0).astype(I32)
    return c0, c1, w0, w1, inb0, inb1


def _prep_body(q_ref, rx_ref, ry_ref, rz_ref,
               ow0, ob0, ww0, wb0, ow1, ob1, ww1, wb1, ow2, ob2, ww2, wb2,
               idx0_o, w0_o, idx1_o, w1_o, idx2_o, w2_o):
    b = pl.program_id(0) // (NBLK // 2)
    q = q_ref[...]
    rx = rx_ref[...]
    ry = ry_ref[...]
    rz = rz_ref[...]
    scales = ((ow0, ob0, ww0, wb0, idx0_o, w0_o),
              (ow1, ob1, ww1, wb1, idx1_o, w1_o),
              (ow2, ob2, ww2, wb2, idx2_o, w2_o))
    for s in range(3):
        ow, ob, ww, wb, idx_o, w_o = scales[s]
        D, H, W = SCALE_DIMS[s]
        off = jnp.tanh(jnp.dot(q, ow[...], preferred_element_type=F32)
                       + ob[...]) * 0.25
        logits = jnp.dot(q, ww[...], preferred_element_type=F32) + wb[...]
        logits = logits - jnp.max(logits, axis=-1, keepdims=True)
        e = jnp.exp(logits)
        attn = e / jnp.sum(e, axis=-1, keepdims=True)
        gx = jnp.clip(rx + off[:, 0:8], 0.0, 1.0) * 2.0 - 1.0
        gy = jnp.clip(ry + off[:, 8:16], 0.0, 1.0) * 2.0 - 1.0
        gz = jnp.clip(rz + off[:, 16:24], 0.0, 1.0) * 2.0 - 1.0
        cx0, cx1, wx0, wx1, ibx0, ibx1 = _corner_terms(gx, float(W))
        cy0, cy1, wy0, wy1, iby0, iby1 = _corner_terms(gy, float(H))
        cz0, cz1, wz0, wz1, ibz0, ibz1 = _corner_terms(gz, float(D))
        idx_cols = []
        w_cols = []
        for cz, czi, wz, ibz in ((0, cz0, wz0, ibz0), (1, cz1, wz1, ibz1)):
            for cy, cyi, wy, iby in ((0, cy0, wy0, iby0), (1, cy1, wy1, iby1)):
                for cx, cxi, wx, ibx in ((0, cx0, wx0, ibx0),
                                         (1, cx1, wx1, ibx1)):
                    idx_cols.append((czi * H + cyi) * W + cxi)
                    w_cols.append(wx * wy * wz * ibx * iby * ibz * attn)
        idx_all = jnp.concatenate(idx_cols, axis=-1) + b * (D * H * W)
        idx_o[...] = idx_all
        w_o[...] = jnp.concatenate(w_cols, axis=-1)


def _full(shape):
    return pl.BlockSpec(shape, lambda i: (0,) * len(shape))


_prep_call = pl.pallas_call(
    _prep_body,
    grid=(NBLK,),
    in_specs=[
        pl.BlockSpec((BQ, 256), lambda i: (i, 0)),
        pl.BlockSpec((BQ, 1), lambda i: (i, 0)),
        pl.BlockSpec((BQ, 1), lambda i: (i, 0)),
        pl.BlockSpec((BQ, 1), lambda i: (i, 0)),
    ] + [_full(s) for s in ((256, 24), (1, 24), (256, 8), (1, 8))] * 3,
    out_specs=[pl.BlockSpec((BQ, 64), lambda i: (i, 0))] * 6,
    out_shape=[jax.ShapeDtypeStruct((NQ, 64), I32),
               jax.ShapeDtypeStruct((NQ, 64), F32)] * 3,
)


def _make_sc_gather(n_rows, C):
    G = C // 32
    mesh = plsc.VectorSubcoreMesh(core_axis_name="c", subcore_axis_name="s",
                                  num_cores=NC)

    @functools.partial(
        pl.kernel, mesh=mesh,
        out_type=jax.ShapeDtypeStruct((NQ, C), F32),
        compiler_params=pltpu.CompilerParams(use_tc_tiling_on_sc=False,
                                             needs_layout_passes=False),
        scratch_types=[
            pltpu.VMEM((2, 2, 128), I32),
            pltpu.VMEM((2, ROWS), F32),
            pltpu.VMEM((2, ROWS, C), F32),
            pltpu.VMEM((2, CQ, C), F32),
            pltpu.SemaphoreType.DMA,
            pltpu.SemaphoreType.DMA,
        ],
    )
    def sc_gather(table, idx_hbm, w_hbm, out_hbm, idxv, wv, rowsv, outv,
                  gsem_a, gsem_b):
        wid = lax.axis_index("s") * NC + lax.axis_index("c")
        q0 = wid * QPW
        gsems = (gsem_a, gsem_b)

        def load_chunk(i, buf):
            rrow = (q0 + i * CQ) * 64 // 128
            pltpu.sync_copy(idx_hbm.at[pl.ds(rrow, 2)], idxv.at[buf])
            pltpu.sync_copy(w_hbm.at[pl.ds((q0 + i * CQ) * 64, ROWS)],
                            wv.at[buf])

        def fire(buf):
            pltpu.async_copy(table.at[idxv.at[buf, 0]],
                             rowsv.at[buf, pl.ds(0, 128)], gsems[buf])
            pltpu.async_copy(table.at[idxv.at[buf, 1]],
                             rowsv.at[buf, pl.ds(128, 128)], gsems[buf])

        def drain(buf):
            pltpu.make_async_copy(table.at[pl.ds(0, ROWS)],
                                  rowsv.at[buf], gsems[buf]).wait()

        def compute(i, buf):
            def qstep(qq, carry):
                def jstep(j16, acc):
                    rb = qq * 64 + j16 * 16
                    wvec = wv[buf, pl.ds(rb, 16)]
                    acc = list(acc)
                    for jj in range(16):
                        wj = wvec[jj]
                        for c in range(2 * G):
                            acc[c] = acc[c] + wj * rowsv[buf, rb + jj,
                                                         pl.ds(c * 16, 16)]
                    return tuple(acc)
                acc = plsc.parallel_loop(
                    0, 4, unroll=2,
                    carry=tuple(jnp.zeros((16,), F32)
                                for _ in range(2 * G)))(jstep)
                for c in range(2 * G):
                    outv[buf, qq, pl.ds(c * 16, 16)] = acc[c]
                return carry
            lax.fori_loop(0, CQ, qstep, 0)
            pltpu.sync_copy(outv.at[buf],
                            out_hbm.at[pl.ds(q0 + i * CQ, CQ)])

        load_chunk(0, 0)
        fire(0)
        load_chunk(1, 1)
        fire(1)

        def step(t, carry):
            i = 2 * t
            drain(0)
            compute(i, 0)
            load_chunk(i + 2, 0)
            fire(0)
            drain(1)
            compute(i + 1, 1)
            load_chunk(i + 3, 1)
            fire(1)
            return carry

        lax.fori_loop(0, NSTEP // 2 - 1, step, 0)
        drain(0)
        compute(NSTEP - 2, 0)
        drain(1)
        compute(NSTEP - 1, 1)

    return sc_gather


@functools.lru_cache(maxsize=None)
def _sc_gather_call(s):
    D, H, W = SCALE_DIMS[s]
    return _make_sc_gather(2 * D * H * W, CHANNELS[s])


def _proj_body(g0_ref, g1_ref, g2_ref,
               vw0, vb0, vw1, vb1, vw2, vb2, lng, lnb, outw, outb, o_ref):
    a0 = jnp.dot(g0_ref[...], vw0[...], preferred_element_type=F32) + vb0[...]
    a1 = jnp.dot(g1_ref[...], vw1[...], preferred_element_type=F32) + vb1[...]
    a2 = jnp.dot(g2_ref[...], vw2[...], preferred_element_type=F32) + vb2[...]
    cat = jnp.concatenate([a0, a1, a2], axis=-1)
    mu = jnp.mean(cat, axis=-1, keepdims=True)
    var = jnp.mean((cat - mu) ** 2, axis=-1, keepdims=True)
    ln = (cat - mu) * jax.lax.rsqrt(var + 1e-5) * lng[...] + lnb[...]
    o_ref[...] = jnp.dot(ln, outw[...], preferred_element_type=F32) + outb[...]


_proj_call = pl.pallas_call(
    _proj_body,
    grid=(NBLK,),
    in_specs=[
        pl.BlockSpec((BQ, 64), lambda i: (i, 0)),
        pl.BlockSpec((BQ, 128), lambda i: (i, 0)),
        pl.BlockSpec((BQ, 128), lambda i: (i, 0)),
        _full((64, 256)), _full((1, 256)),
        _full((128, 256)), _full((1, 256)),
        _full((128, 256)), _full((1, 256)),
        _full((1, 768)), _full((1, 768)),
        _full((768, 256)), _full((1, 256)),
    ],
    out_specs=pl.BlockSpec((BQ, 256), lambda i: (i, 0)),
    out_shape=jax.ShapeDtypeStruct((NQ, 256), F32),
)


def _xpose_body(v_ref, o_ref):
    o_ref[...] = jnp.swapaxes(v_ref[0], 0, 1)[None].astype(jnp.bfloat16)


@functools.lru_cache(maxsize=None)
def _xpose_call(s):
    D, H, W = SCALE_DIMS[s]
    C = CHANNELS[s]
    DHW = D * H * W
    CH_ROWS = 2048
    K = DHW // CH_ROWS
    return pl.pallas_call(
        _xpose_body,
        grid=(2, K),
        in_specs=[pl.BlockSpec((1, C, CH_ROWS), lambda b, k: (b, 0, k))],
        out_specs=pl.BlockSpec((1, CH_ROWS, C), lambda b, k: (b, k, 0)),
        out_shape=jax.ShapeDtypeStruct((2, DHW, C), jnp.bfloat16),
    )


def _unpack_perm(C):
    perm = []
    for g in range(C // 32):
        perm += [32 * g + 2 * k for k in range(16)]
        perm += [32 * g + 2 * k + 1 for k in range(16)]
    return perm


def _reorder_off(w):
    return w.reshape(-1, 8, 3).swapaxes(-1, -2).reshape(w.shape[:-1] + (24,))


def kernel(queries, reference_points, occ_vol_0, occ_vol_1, occ_vol_2,
           off_w0, off_b0, wt_w0, wt_b0, val_w0, val_b0,
           off_w1, off_b1, wt_w1, wt_b1, val_w1, val_b1,
           off_w2, off_b2, wt_w2, wt_b2, val_w2, val_b2,
           ln_g, ln_b, out_w, out_b):
    q2d = queries.reshape(NQ, 256)
    ref2d = reference_points.reshape(NQ, 3)
    rx, ry, rz = ref2d[:, 0:1], ref2d[:, 1:2], ref2d[:, 2:3]
    prep_args = [q2d, rx, ry, rz]
    for ow, ob, ww, wb in ((off_w0, off_b0, wt_w0, wt_b0),
                           (off_w1, off_b1, wt_w1, wt_b1),
                           (off_w2, off_b2, wt_w2, wt_b2)):
        prep_args += [_reorder_off(ow), _reorder_off(ob.reshape(1, 24)),
                      ww, wb.reshape(1, 8)]
    idx0, w0, idx1, w1, idx2, w2 = _prep_call(*prep_args)

    gathered = []
    for s, (vol, idx, w) in enumerate(((occ_vol_0, idx0, w0),
                                       (occ_vol_1, idx1, w1),
                                       (occ_vol_2, idx2, w2))):
        D, H, W = SCALE_DIMS[s]
        C = CHANNELS[s]
        table = vol.transpose(0, 2, 3, 4, 1).reshape(2 * D * H * W, C)
        gathered.append(_sc_gather_call(s)(table, idx.reshape(-1, 128),
                                           w.reshape(-1)))

    out = _proj_call(gathered[0], gathered[1], gathered[2],
                     val_w0, val_b0.reshape(1, 256),
                     val_w1, val_b1.reshape(1, 256),
                     val_w2, val_b2.reshape(1, 256),
                     ln_g.reshape(1, 768), ln_b.reshape(1, 768),
                     out_w, out_b.reshape(1, 256))
    return out.reshape(2, 4096, 256)


# speedup vs baseline: 1.5247x; 1.0827x over previous
import functools

import jax
import jax.numpy as jnp
from jax import lax
from jax.experimental import pallas as pl
from jax.experimental.pallas import tpu as pltpu
from jax.experimental.pallas import tpu_sc as plsc

F32 = jnp.float32
I32 = jnp.int32

NQ = 8192
BQ = 512
NBLK = NQ // BQ
SCALE_DIMS = ((16, 128, 128), (8, 64, 64), (4, 32, 32))
CHANNELS = (64, 128, 128)

NC, NS = 2, 16
NW = NC * NS
QPW = NQ // NW
CQ = 4
ROWS = CQ * 64
NSTEP = QPW // CQ


def _corner_terms(g, dim):
    ix = ((g + 1.0) * dim - 1.0) * 0.5
    i0 = jnp.floor(ix)
    w1 = ix - i0
    w0 = 1.0 - w1
    i1 = i0 + 1.0
    inb0 = ((i0 >= 0.0) & (i0 <= dim - 1.0)).astype(F32)
    inb1 = ((i1 >= 0.0) & (i1 <= dim - 1.0)).astype(F32)
    c0 = jnp.clip(i0, 0.0, dim - 1.0).astype(I32)
    c1 = jnp.clip(i1, 0.0, dim - 1.0).astype(I32)
    return c0, c1, w0, w1, inb0, inb1


def _prep_body(q_ref, rx_ref, ry_ref, rz_ref,
               ow0, ob0, ww0, wb0, ow1, ob1, ww1, wb1, ow2, ob2, ww2, wb2,
               idx0_o, w0_o, idx1_o, w1_o, idx2_o, w2_o):
    b = pl.program_id(0) // (NBLK // 2)
    q = q_ref[...]
    rx = rx_ref[...]
    ry = ry_ref[...]
    rz = rz_ref[...]
    scales = ((ow0, ob0, ww0, wb0, idx0_o, w0_o),
              (ow1, ob1, ww1, wb1, idx1_o, w1_o),
              (ow2, ob2, ww2, wb2, idx2_o, w2_o))
    for s in range(3):
        ow, ob, ww, wb, idx_o, w_o = scales[s]
        D, H, W = SCALE_DIMS[s]
        off = jnp.tanh(jnp.dot(q, ow[...], preferred_element_type=F32)
                       + ob[...]) * 0.25
        logits = jnp.dot(q, ww[...], preferred_element_type=F32) + wb[...]
        logits = logits - jnp.max(logits, axis=-1, keepdims=True)
        e = jnp.exp(logits)
        attn = e / jnp.sum(e, axis=-1, keepdims=True)
        gx = jnp.clip(rx + off[:, 0:8], 0.0, 1.0) * 2.0 - 1.0
        gy = jnp.clip(ry + off[:, 8:16], 0.0, 1.0) * 2.0 - 1.0
        gz = jnp.clip(rz + off[:, 16:24], 0.0, 1.0) * 2.0 - 1.0
        cx0, cx1, wx0, wx1, ibx0, ibx1 = _corner_terms(gx, float(W))
        cy0, cy1, wy0, wy1, iby0, iby1 = _corner_terms(gy, float(H))
        cz0, cz1, wz0, wz1, ibz0, ibz1 = _corner_terms(gz, float(D))
        idx_cols = []
        w_cols = []
        for cz, czi, wz, ibz in ((0, cz0, wz0, ibz0), (1, cz1, wz1, ibz1)):
            for cy, cyi, wy, iby in ((0, cy0, wy0, iby0), (1, cy1, wy1, iby1)):
                for cx, cxi, wx, ibx in ((0, cx0, wx0, ibx0),
                                         (1, cx1, wx1, ibx1)):
                    idx_cols.append((czi * H + cyi) * W + cxi)
                    w_cols.append(wx * wy * wz * ibx * iby * ibz * attn)
        idx_all = jnp.concatenate(idx_cols, axis=-1) + b * (D * H * W)
        idx_o[...] = idx_all
        w_o[...] = jnp.concatenate(w_cols, axis=-1)


def _full(shape):
    return pl.BlockSpec(shape, lambda i: (0,) * len(shape))


_prep_call = pl.pallas_call(
    _prep_body,
    grid=(NBLK,),
    in_specs=[
        pl.BlockSpec((BQ, 256), lambda i: (i, 0)),
        pl.BlockSpec((BQ, 1), lambda i: (i, 0)),
        pl.BlockSpec((BQ, 1), lambda i: (i, 0)),
        pl.BlockSpec((BQ, 1), lambda i: (i, 0)),
    ] + [_full(s) for s in ((256, 24), (1, 24), (256, 8), (1, 8))] * 3,
    out_specs=[pl.BlockSpec((BQ, 64), lambda i: (i, 0))] * 6,
    out_shape=[jax.ShapeDtypeStruct((NQ, 64), I32),
               jax.ShapeDtypeStruct((NQ, 64), F32)] * 3,
)


def _make_sc_gather(n_rows, C):
    G = C // 32
    mesh = plsc.VectorSubcoreMesh(core_axis_name="c", subcore_axis_name="s",
                                  num_cores=NC)

    @functools.partial(
        pl.kernel, mesh=mesh,
        out_type=jax.ShapeDtypeStruct((NQ, C), F32),
        compiler_params=pltpu.CompilerParams(use_tc_tiling_on_sc=False,
                                             needs_layout_passes=False),
        scratch_types=[
            pltpu.VMEM((2, 2, 128), I32),
            pltpu.VMEM((2, ROWS), F32),
            pltpu.VMEM((2, ROWS, C), F32),
            pltpu.VMEM((2, CQ, C), F32),
            pltpu.SemaphoreType.DMA,
            pltpu.SemaphoreType.DMA,
        ],
    )
    def sc_gather(table, idx_hbm, w_hbm, out_hbm, idxv, wv, rowsv, outv,
                  gsem_a, gsem_b):
        wid = lax.axis_index("s") * NC + lax.axis_index("c")
        q0 = wid * QPW
        gsems = (gsem_a, gsem_b)

        def load_chunk(i, buf):
            rrow = (q0 + i * CQ) * 64 // 128
            pltpu.sync_copy(idx_hbm.at[pl.ds(rrow, 2)], idxv.at[buf])
            pltpu.sync_copy(w_hbm.at[pl.ds((q0 + i * CQ) * 64, ROWS)],
                            wv.at[buf])

        def fire(buf):
            pltpu.async_copy(table.at[idxv.at[buf, 0]],
                             rowsv.at[buf, pl.ds(0, 128)], gsems[buf])
            pltpu.async_copy(table.at[idxv.at[buf, 1]],
                             rowsv.at[buf, pl.ds(128, 128)], gsems[buf])

        def drain(buf):
            pltpu.make_async_copy(table.at[pl.ds(0, ROWS)],
                                  rowsv.at[buf], gsems[buf]).wait()

        def compute(i, buf):
            def qstep(qq, carry):
                def jstep(j16, acc):
                    rb = qq * 64 + j16 * 16
                    wvec = wv[buf, pl.ds(rb, 16)]
                    acc = list(acc)
                    for jj in range(16):
                        wj = wvec[jj]
                        for c in range(2 * G):
                            acc[c] = acc[c] + wj * rowsv[buf, rb + jj,
                                                         pl.ds(c * 16, 16)]
                    return tuple(acc)
                acc = lax.fori_loop(
                    0, 4, jstep,
                    tuple(jnp.zeros((16,), F32) for _ in range(2 * G)))
                for c in range(2 * G):
                    outv[buf, qq, pl.ds(c * 16, 16)] = acc[c]
                return carry
            lax.fori_loop(0, CQ, qstep, 0)
            pltpu.sync_copy(outv.at[buf],
                            out_hbm.at[pl.ds(q0 + i * CQ, CQ)])

        load_chunk(0, 0)
        fire(0)
        load_chunk(1, 1)
        fire(1)

        def step(t, carry):
            i = 2 * t
            drain(0)
            compute(i, 0)
            load_chunk(i + 2, 0)
            fire(0)
            drain(1)
            compute(i + 1, 1)
            load_chunk(i + 3, 1)
            fire(1)
            return carry

        lax.fori_loop(0, NSTEP // 2 - 1, step, 0)
        drain(0)
        compute(NSTEP - 2, 0)
        drain(1)
        compute(NSTEP - 1, 1)

    return sc_gather


@functools.lru_cache(maxsize=None)
def _sc_gather_call(s):
    D, H, W = SCALE_DIMS[s]
    return _make_sc_gather(2 * D * H * W, CHANNELS[s])


def _proj_body(g0_ref, g1_ref, g2_ref,
               vw0, vb0, vw1, vb1, vw2, vb2, lng, lnb, outw, outb, o_ref):
    a0 = jnp.dot(g0_ref[...], vw0[...], preferred_element_type=F32) + vb0[...]
    a1 = jnp.dot(g1_ref[...], vw1[...], preferred_element_type=F32) + vb1[...]
    a2 = jnp.dot(g2_ref[...], vw2[...], preferred_element_type=F32) + vb2[...]
    cat = jnp.concatenate([a0, a1, a2], axis=-1)
    mu = jnp.mean(cat, axis=-1, keepdims=True)
    var = jnp.mean((cat - mu) ** 2, axis=-1, keepdims=True)
    ln = (cat - mu) * jax.lax.rsqrt(var + 1e-5) * lng[...] + lnb[...]
    o_ref[...] = jnp.dot(ln, outw[...], preferred_element_type=F32) + outb[...]


_proj_call = pl.pallas_call(
    _proj_body,
    grid=(NBLK,),
    in_specs=[
        pl.BlockSpec((BQ, 64), lambda i: (i, 0)),
        pl.BlockSpec((BQ, 128), lambda i: (i, 0)),
        pl.BlockSpec((BQ, 128), lambda i: (i, 0)),
        _full((64, 256)), _full((1, 256)),
        _full((128, 256)), _full((1, 256)),
        _full((128, 256)), _full((1, 256)),
        _full((1, 768)), _full((1, 768)),
        _full((768, 256)), _full((1, 256)),
    ],
    out_specs=pl.BlockSpec((BQ, 256), lambda i: (i, 0)),
    out_shape=jax.ShapeDtypeStruct((NQ, 256), F32),
)


def _xpose_body(v_ref, o_ref):
    o_ref[...] = jnp.swapaxes(v_ref[0], 0, 1)[None].astype(jnp.bfloat16)


@functools.lru_cache(maxsize=None)
def _xpose_call(s):
    D, H, W = SCALE_DIMS[s]
    C = CHANNELS[s]
    DHW = D * H * W
    CH_ROWS = 2048
    K = DHW // CH_ROWS
    return pl.pallas_call(
        _xpose_body,
        grid=(2, K),
        in_specs=[pl.BlockSpec((1, C, CH_ROWS), lambda b, k: (b, 0, k))],
        out_specs=pl.BlockSpec((1, CH_ROWS, C), lambda b, k: (b, k, 0)),
        out_shape=jax.ShapeDtypeStruct((2, DHW, C), jnp.bfloat16),
    )


def _unpack_perm(C):
    perm = []
    for g in range(C // 32):
        perm += [32 * g + 2 * k for k in range(16)]
        perm += [32 * g + 2 * k + 1 for k in range(16)]
    return perm


def _reorder_off(w):
    return w.reshape(-1, 8, 3).swapaxes(-1, -2).reshape(w.shape[:-1] + (24,))


def kernel(queries, reference_points, occ_vol_0, occ_vol_1, occ_vol_2,
           off_w0, off_b0, wt_w0, wt_b0, val_w0, val_b0,
           off_w1, off_b1, wt_w1, wt_b1, val_w1, val_b1,
           off_w2, off_b2, wt_w2, wt_b2, val_w2, val_b2,
           ln_g, ln_b, out_w, out_b):
    q2d = queries.reshape(NQ, 256)
    ref2d = reference_points.reshape(NQ, 3)
    rx, ry, rz = ref2d[:, 0:1], ref2d[:, 1:2], ref2d[:, 2:3]
    prep_args = [q2d, rx, ry, rz]
    for ow, ob, ww, wb in ((off_w0, off_b0, wt_w0, wt_b0),
                           (off_w1, off_b1, wt_w1, wt_b1),
                           (off_w2, off_b2, wt_w2, wt_b2)):
        prep_args += [_reorder_off(ow), _reorder_off(ob.reshape(1, 24)),
                      ww, wb.reshape(1, 8)]
    idx0, w0, idx1, w1, idx2, w2 = _prep_call(*prep_args)

    gathered = []
    for s, (vol, idx, w) in enumerate(((occ_vol_0, idx0, w0),
                                       (occ_vol_1, idx1, w1),
                                       (occ_vol_2, idx2, w2))):
        D, H, W = SCALE_DIMS[s]
        C = CHANNELS[s]
        table = vol.transpose(0, 2, 3, 4, 1).reshape(2 * D * H * W, C)
        gathered.append(_sc_gather_call(s)(table, idx.reshape(-1, 128),
                                           w.reshape(-1)))

    out = _proj_call(gathered[0], gathered[1], gathered[2],
                     val_w0, val_b0.reshape(1, 256),
                     val_w1, val_b1.reshape(1, 256),
                     val_w2, val_b2.reshape(1, 256),
                     ln_g.reshape(1, 768), ln_b.reshape(1, 768),
                     out_w, out_b.reshape(1, 256))
    return out.reshape(2, 4096, 256)
